# baseline (speedup 1.0000x reference)
import functools

import jax
import jax.numpy as jnp
from jax import lax
from jax.experimental import pallas as pl
from jax.experimental.pallas import tpu as pltpu
from jax.experimental.pallas import tpu_sc as plsc

D = 768
E = 8
H = 4 * D
S = 2048
T = 256
K = 15
P = K * T
C = 512
NCH = S // C

NC = 2
NS = 16
NW = NC * NS
CH = S // NW


def _gelu(x):
    return x * 0.5 * (1.0 + lax.erf(x * 0.7071067811865476))


def _router_body(x_ref, wr_ref, probs_ref, dst_ref, te_ref, ta_ref):
    logits = lax.dot_general(
        x_ref[...], wr_ref[...], (((1,), (1,)), ((), ())),
        preferred_element_type=jnp.float32)
    m = jnp.max(logits, axis=-1, keepdims=True)
    ex = jnp.exp(logits - m)
    p = ex / jnp.sum(ex, axis=-1, keepdims=True)
    probs_ref[...] = jnp.max(p, axis=-1, keepdims=True)
    eidx = jnp.argmax(p, axis=-1).astype(jnp.int32).reshape(-1, 1)

    lane = lax.broadcasted_iota(jnp.int32, (S, E), 1)
    onehot = (lane == eidx).astype(jnp.float32)

    counts = jnp.sum(onehot, axis=0, keepdims=True)
    pc_i = ((counts.astype(jnp.int32) + T - 1) // T) * T
    pc_f = pc_i.astype(jnp.float32)
    er = lax.broadcasted_iota(jnp.int32, (E, E), 0)
    ec = lax.broadcasted_iota(jnp.int32, (E, E), 1)
    strict8 = (er < ec).astype(jnp.float32)
    off_f = lax.dot_general(pc_f, strict8, (((1,), (0,)), ((), ())),
                            preferred_element_type=jnp.float32)
    total_f = off_f[:, E - 1:E] + pc_f[:, E - 1:E]
    total_i = total_f.astype(jnp.int32)

    rr = lax.broadcasted_iota(jnp.int32, (C, C), 0)
    cc = lax.broadcasted_iota(jnp.int32, (C, C), 1)
    trilC = (cc < rr).astype(jnp.float32)

    carry = jnp.zeros((1, E), jnp.float32)
    dst_parts = []
    for n in range(NCH):
        oh_n = lax.slice(onehot, (n * C, 0), ((n + 1) * C, E))
        within = lax.dot_general(trilC, oh_n, (((1,), (0,)), ((), ())),
                                 preferred_element_type=jnp.float32)
        slot = jnp.sum((within + carry + off_f) * oh_n, axis=-1,
                       keepdims=True)
        dst_parts.append(slot)
        carry = carry + jnp.sum(oh_n, axis=0, keepdims=True)
    dst_ref[...] = jnp.concatenate(dst_parts, axis=0).astype(jnp.int32)

    kT = lax.broadcasted_iota(jnp.int32, (1, K), 1) * T
    kTc = jnp.minimum(kT, total_i - 1).astype(jnp.float32)
    ends_f = off_f + pc_f
    te = jnp.zeros((1, K), jnp.int32)
    for e in range(E):
        end_e = lax.slice(ends_f, (0, e), (1, e + 1))
        te = te + (kTc >= end_e).astype(jnp.int32)
    te_ref[...] = te
    ta_ref[...] = (kT < total_i).astype(jnp.int32)


def _mlp_body(te_ref, ta_ref, x_ref, w1_ref, b1_ref, w2_ref, b2_ref, o_ref):
    k = pl.program_id(0)
    j = pl.program_id(1)

    @pl.when(ta_ref[k] == 1)
    def _():
        x = x_ref[...]
        h = lax.dot_general(
            x, w1_ref[0], (((1,), (1,)), ((), ())),
            preferred_element_type=jnp.float32) + b1_ref[0]
        h = _gelu(h)
        part = lax.dot_general(
            h, w2_ref[0], (((1,), (1,)), ((), ())),
            preferred_element_type=jnp.float32)

        @pl.when(j == 0)
        def _():
            o_ref[...] = part + b2_ref[0]

        @pl.when(j > 0)
        def _():
            o_ref[...] = o_ref[...] + part


def _scale_body(p_ref, y_ref, o_ref):
    o_ref[...] = p_ref[...] * y_ref[...]


def _wcast_body(w1_ref, w2_ref, o1_ref, o2_ref):
    o1_ref[...] = w1_ref[...].astype(jnp.bfloat16)
    o2_ref[...] = w2_ref[...].astype(jnp.bfloat16)


def _wcast(fc1_w, fc2_w):
    G = 12
    w1f = fc1_w.reshape(E * H, D)
    w2f = fc2_w.reshape(E * D, H)
    r1 = (E * H) // G
    r2 = (E * D) // G
    w1b, w2b = pl.pallas_call(
        _wcast_body,
        grid=(G,),
        in_specs=[
            pl.BlockSpec((r1, D), lambda i: (i, 0)),
            pl.BlockSpec((r2, H), lambda i: (i, 0)),
        ],
        out_specs=[
            pl.BlockSpec((r1, D), lambda i: (i, 0)),
            pl.BlockSpec((r2, H), lambda i: (i, 0)),
        ],
        out_shape=[
            jax.ShapeDtypeStruct((E * H, D), jnp.bfloat16),
            jax.ShapeDtypeStruct((E * D, H), jnp.bfloat16),
        ],
        compiler_params=pltpu.CompilerParams(
            dimension_semantics=("arbitrary",)),
    )(w1f, w2f)
    return w1b.reshape(E, H, D), w2b.reshape(E, D, H)


@functools.lru_cache(maxsize=None)
def _sc_kernels():
    mesh = plsc.VectorSubcoreMesh(core_axis_name="c", subcore_axis_name="s")
    scratch = [
        pltpu.VMEM((CH,), jnp.int32),
        pltpu.VMEM((CH, D), jnp.float32),
        pltpu.SemaphoreType.DMA,
    ]

    @functools.partial(
        pl.kernel, mesh=mesh,
        out_type=jax.ShapeDtypeStruct((P, D), jnp.float32),
        scratch_types=scratch)
    def dispatch(x_hbm, dst_hbm, xs_hbm, idx_v, rows_v, sem):
        wid = lax.axis_index("s") * NC + lax.axis_index("c")
        base = wid * CH
        pltpu.sync_copy(dst_hbm.at[pl.ds(base, CH)], idx_v)
        pltpu.sync_copy(x_hbm.at[pl.ds(base, CH)], rows_v)
        pltpu.async_copy(rows_v, xs_hbm.at[idx_v], sem).wait()

    @functools.partial(
        pl.kernel, mesh=mesh,
        out_type=jax.ShapeDtypeStruct((S, D), jnp.float32),
        scratch_types=scratch)
    def combine(ys_hbm, dst_hbm, out_hbm, idx_v, rows_v, sem):
        wid = lax.axis_index("s") * NC + lax.axis_index("c")
        base = wid * CH
        pltpu.sync_copy(dst_hbm.at[pl.ds(base, CH)], idx_v)
        pltpu.async_copy(ys_hbm.at[idx_v], rows_v, sem).wait()
        pltpu.sync_copy(rows_v, out_hbm.at[pl.ds(base, CH)])

    return dispatch, combine


def _dispatch_sc(x, dst):
    return _sc_kernels()[0](x, dst)


def _combine_sc(ys, dst):
    return _sc_kernels()[1](ys, dst)


def _router(x, Wr):
    return pl.pallas_call(
        _router_body,
        out_shape=[
            jax.ShapeDtypeStruct((S, 1), jnp.float32),
            jax.ShapeDtypeStruct((S, 1), jnp.int32),
            jax.ShapeDtypeStruct((1, K), jnp.int32),
            jax.ShapeDtypeStruct((1, K), jnp.int32),
        ],
    )(x, Wr)


JH = 4
H2 = H // JH


def _grouped_mlp(te, ta, xs, fc1_w, fc1_b, fc2_w, fc2_b):
    grid_spec = pltpu.PrefetchScalarGridSpec(
        num_scalar_prefetch=2,
        grid=(K, JH),
        in_specs=[
            pl.BlockSpec((T, D), lambda k, j, te, ta: (k, 0)),
            pl.BlockSpec((1, H2, D), lambda k, j, te, ta: (te[k], j, 0)),
            pl.BlockSpec((1, 1, H2), lambda k, j, te, ta: (te[k], 0, j)),
            pl.BlockSpec((1, D, H2), lambda k, j, te, ta: (te[k], 0, j)),
            pl.BlockSpec((1, 1, D), lambda k, j, te, ta: (te[k], 0, 0)),
        ],
        out_specs=pl.BlockSpec((T, D), lambda k, j, te, ta: (k, 0)),
    )
    return pl.pallas_call(
        _mlp_body,
        grid_spec=grid_spec,
        out_shape=jax.ShapeDtypeStruct((P, D), jnp.float32),
        compiler_params=pltpu.CompilerParams(
            dimension_semantics=("arbitrary", "arbitrary")),
    )(te, ta, xs, fc1_w, fc1_b.reshape(E, 1, H), fc2_w,
      fc2_b.reshape(E, 1, D))


def _scale(probs, y):
    return pl.pallas_call(
        _scale_body,
        out_shape=jax.ShapeDtypeStruct((S, D), jnp.float32),
    )(probs, y)


def kernel(hidden_states, Wr, fc1_w, fc1_b, fc2_w, fc2_b):
    B = hidden_states.shape[0]
    x = hidden_states.reshape(S, D)
    probs, dst2d, te2d, ta2d = _router(x, Wr)
    w1b, w2b = _wcast(fc1_w, fc2_w)
    dst = dst2d.reshape(S)
    xs = _dispatch_sc(x, dst)
    ys = _grouped_mlp(te2d.reshape(K), ta2d.reshape(K), xs,
                      w1b, fc1_b, w2b, fc2_b)
    y = _combine_sc(ys, dst)
    return _scale(probs, y).reshape(B, S, D)


# speedup vs baseline: 1.7407x; 1.7407x over previous
import functools

import jax
import jax.numpy as jnp
from jax import lax
from jax.experimental import pallas as pl
from jax.experimental.pallas import tpu as pltpu
from jax.experimental.pallas import tpu_sc as plsc

D = 768
E = 8
H = 4 * D
S = 2048
T = 256
K = 15
P = K * T
C = 512
NCH = S // C

NC = 2
NS = 16
NW = NC * NS
CH = S // NW


def _gelu(x):
    return x * 0.5 * (1.0 + lax.erf(x * 0.7071067811865476))


def _router_body(x_ref, wr_ref, probs_ref, dst_ref, te_ref, ta_ref):
    logits = lax.dot_general(
        x_ref[...], wr_ref[...], (((1,), (1,)), ((), ())),
        preferred_element_type=jnp.float32)
    m = jnp.max(logits, axis=-1, keepdims=True)
    ex = jnp.exp(logits - m)
    p = ex / jnp.sum(ex, axis=-1, keepdims=True)
    probs_ref[...] = jnp.broadcast_to(jnp.max(p, axis=-1, keepdims=True),
                                      (S, 128))
    eidx = jnp.argmax(p, axis=-1).astype(jnp.int32).reshape(-1, 1)

    lane = lax.broadcasted_iota(jnp.int32, (S, E), 1)
    onehot = (lane == eidx).astype(jnp.float32)

    counts = jnp.sum(onehot, axis=0, keepdims=True)
    pc_i = ((counts.astype(jnp.int32) + T - 1) // T) * T
    pc_f = pc_i.astype(jnp.float32)
    er = lax.broadcasted_iota(jnp.int32, (E, E), 0)
    ec = lax.broadcasted_iota(jnp.int32, (E, E), 1)
    strict8 = (er < ec).astype(jnp.float32)
    off_f = lax.dot_general(pc_f, strict8, (((1,), (0,)), ((), ())),
                            preferred_element_type=jnp.float32)
    total_f = off_f[:, E - 1:E] + pc_f[:, E - 1:E]
    total_i = total_f.astype(jnp.int32)

    rr = lax.broadcasted_iota(jnp.int32, (C, C), 0)
    cc = lax.broadcasted_iota(jnp.int32, (C, C), 1)
    trilC = (cc < rr).astype(jnp.float32)

    carry = jnp.zeros((1, E), jnp.float32)
    dst_parts = []
    for n in range(NCH):
        oh_n = lax.slice(onehot, (n * C, 0), ((n + 1) * C, E))
        within = lax.dot_general(trilC, oh_n, (((1,), (0,)), ((), ())),
                                 preferred_element_type=jnp.float32)
        slot = jnp.sum((within + carry + off_f) * oh_n, axis=-1,
                       keepdims=True)
        dst_parts.append(slot)
        carry = carry + jnp.sum(oh_n, axis=0, keepdims=True)
    dst_ref[...] = jnp.concatenate(dst_parts, axis=0).astype(jnp.int32)

    kT = lax.broadcasted_iota(jnp.int32, (1, K), 1) * T
    kTc = jnp.minimum(kT, total_i - 1).astype(jnp.float32)
    ends_f = off_f + pc_f
    te = jnp.zeros((1, K), jnp.int32)
    for e in range(E):
        end_e = lax.slice(ends_f, (0, e), (1, e + 1))
        te = te + (kTc >= end_e).astype(jnp.int32)
    te_ref[...] = te
    ta_ref[...] = (kT < total_i).astype(jnp.int32)


def _mlp_body(te_ref, ta_ref, ps_ref, x_ref, w1_ref, b1_ref, w2_ref, b2_ref,
              o_ref):
    k = pl.program_id(0)

    @pl.when(ta_ref[k] == 1)
    def _():
        x = x_ref[...]
        h = lax.dot_general(
            x, w1_ref[0], (((1,), (1,)), ((), ())),
            preferred_element_type=jnp.float32) + b1_ref[0]
        h = _gelu(h)
        y = lax.dot_general(
            h, w2_ref[0], (((1,), (1,)), ((), ())),
            preferred_element_type=jnp.float32) + b2_ref[0]
        o_ref[...] = ps_ref[:, :1] * y


@functools.lru_cache(maxsize=None)
def _sc_kernels():
    mesh = plsc.VectorSubcoreMesh(core_axis_name="c", subcore_axis_name="s")
    scratch = [
        pltpu.VMEM((CH,), jnp.int32),
        pltpu.VMEM((CH, D), jnp.float32),
        pltpu.SemaphoreType.DMA,
    ]

    @functools.partial(
        pl.kernel, mesh=mesh,
        out_type=[jax.ShapeDtypeStruct((P, D), jnp.float32),
                  jax.ShapeDtypeStruct((P, 128), jnp.float32)],
        scratch_types=scratch + [pltpu.VMEM((CH, 128), jnp.float32)])
    def dispatch(x_hbm, pw_hbm, dst_hbm, xs_hbm, ps_hbm, idx_v, rows_v, sem,
                 pv):
        wid = lax.axis_index("s") * NC + lax.axis_index("c")
        base = wid * CH
        pltpu.sync_copy(dst_hbm.at[pl.ds(base, CH)], idx_v)
        pltpu.sync_copy(x_hbm.at[pl.ds(base, CH)], rows_v)
        pltpu.sync_copy(pw_hbm.at[pl.ds(base, CH)], pv)
        pltpu.async_copy(rows_v, xs_hbm.at[idx_v], sem).wait()
        pltpu.async_copy(pv, ps_hbm.at[idx_v], sem).wait()

    @functools.partial(
        pl.kernel, mesh=mesh,
        out_type=jax.ShapeDtypeStruct((S, D), jnp.float32),
        scratch_types=scratch)
    def combine(ys_hbm, dst_hbm, out_hbm, idx_v, rows_v, sem):
        wid = lax.axis_index("s") * NC + lax.axis_index("c")
        base = wid * CH
        pltpu.sync_copy(dst_hbm.at[pl.ds(base, CH)], idx_v)
        pltpu.async_copy(ys_hbm.at[idx_v], rows_v, sem).wait()
        pltpu.sync_copy(rows_v, out_hbm.at[pl.ds(base, CH)])

    return dispatch, combine


def _dispatch_sc(x, pw, dst):
    return _sc_kernels()[0](x, pw, dst)


def _combine_sc(ys, dst):
    return _sc_kernels()[1](ys, dst)


def _router(x, Wr):
    return pl.pallas_call(
        _router_body,
        out_shape=[
            jax.ShapeDtypeStruct((S, 128), jnp.float32),
            jax.ShapeDtypeStruct((S, 1), jnp.int32),
            jax.ShapeDtypeStruct((1, K), jnp.int32),
            jax.ShapeDtypeStruct((1, K), jnp.int32),
        ],
    )(x, Wr)


JH = 4
H2 = H // JH


def _grouped_mlp(te, ta, ps, xs, fc1_w, fc1_b, fc2_w, fc2_b):
    grid_spec = pltpu.PrefetchScalarGridSpec(
        num_scalar_prefetch=2,
        grid=(K,),
        in_specs=[
            pl.BlockSpec((T, 128), lambda k, te, ta: (k, 0)),
            pl.BlockSpec((T, D), lambda k, te, ta: (k, 0)),
            pl.BlockSpec((1, H, D), lambda k, te, ta: (te[k], 0, 0)),
            pl.BlockSpec((1, 1, H), lambda k, te, ta: (te[k], 0, 0)),
            pl.BlockSpec((1, D, H), lambda k, te, ta: (te[k], 0, 0)),
            pl.BlockSpec((1, 1, D), lambda k, te, ta: (te[k], 0, 0)),
        ],
        out_specs=pl.BlockSpec((T, D), lambda k, te, ta: (k, 0)),
    )
    return pl.pallas_call(
        _mlp_body,
        grid_spec=grid_spec,
        out_shape=jax.ShapeDtypeStruct((P, D), jnp.float32),
        compiler_params=pltpu.CompilerParams(
            dimension_semantics=("arbitrary",)),
    )(te, ta, ps, xs, fc1_w, fc1_b.reshape(E, 1, H), fc2_w,
      fc2_b.reshape(E, 1, D))


def kernel(hidden_states, Wr, fc1_w, fc1_b, fc2_w, fc2_b):
    B = hidden_states.shape[0]
    x = hidden_states.reshape(S, D)
    pw, dst2d, te2d, ta2d = _router(x, Wr)
    dst = dst2d.reshape(S)
    xs, ps = _dispatch_sc(x, pw, dst)
    ys = _grouped_mlp(te2d.reshape(K), ta2d.reshape(K), ps, xs,
                      fc1_w, fc1_b, fc2_w, fc2_b)
    y = _combine_sc(ys, dst)
    return y.reshape(B, S, D)


# speedup vs baseline: 1.7607x; 1.0115x over previous
import functools

import jax
import jax.numpy as jnp
from jax import lax
from jax.experimental import pallas as pl
from jax.experimental.pallas import tpu as pltpu
from jax.experimental.pallas import tpu_sc as plsc

D = 768
E = 8
H = 4 * D
S = 2048
T = 256
K = 15
P = K * T
C = 512
NCH = S // C

NC = 2
NS = 16
NW = NC * NS
CH = S // NW


def _gelu(x):
    return x * 0.5 * (1.0 + lax.erf(x * 0.7071067811865476))


def _router_body(x_ref, wr_ref, probs_ref, dst_ref, te_ref, ta_ref):
    logits = lax.dot_general(
        x_ref[...], wr_ref[...], (((1,), (1,)), ((), ())),
        preferred_element_type=jnp.float32)
    m = jnp.max(logits, axis=-1, keepdims=True)
    ex = jnp.exp(logits - m)
    p = ex / jnp.sum(ex, axis=-1, keepdims=True)
    probs_ref[...] = jnp.broadcast_to(jnp.max(p, axis=-1, keepdims=True),
                                      (S, 128))
    eidx = jnp.argmax(p, axis=-1).astype(jnp.int32).reshape(-1, 1)

    lane = lax.broadcasted_iota(jnp.int32, (S, E), 1)
    onehot = (lane == eidx).astype(jnp.float32)

    counts = jnp.sum(onehot, axis=0, keepdims=True)
    pc_i = ((counts.astype(jnp.int32) + T - 1) // T) * T
    pc_f = pc_i.astype(jnp.float32)
    er = lax.broadcasted_iota(jnp.int32, (E, E), 0)
    ec = lax.broadcasted_iota(jnp.int32, (E, E), 1)
    strict8 = (er < ec).astype(jnp.float32)
    off_f = lax.dot_general(pc_f, strict8, (((1,), (0,)), ((), ())),
                            preferred_element_type=jnp.float32)
    total_f = off_f[:, E - 1:E] + pc_f[:, E - 1:E]
    total_i = total_f.astype(jnp.int32)

    rr = lax.broadcasted_iota(jnp.int32, (C, C), 0)
    cc = lax.broadcasted_iota(jnp.int32, (C, C), 1)
    trilC = (cc < rr).astype(jnp.float32)

    carry = jnp.zeros((1, E), jnp.float32)
    dst_parts = []
    for n in range(NCH):
        oh_n = lax.slice(onehot, (n * C, 0), ((n + 1) * C, E))
        within = lax.dot_general(trilC, oh_n, (((1,), (0,)), ((), ())),
                                 preferred_element_type=jnp.float32)
        slot = jnp.sum((within + carry + off_f) * oh_n, axis=-1,
                       keepdims=True)
        dst_parts.append(slot)
        carry = carry + jnp.sum(oh_n, axis=0, keepdims=True)
    dst_ref[...] = jnp.concatenate(dst_parts, axis=0).astype(jnp.int32)

    kT = lax.broadcasted_iota(jnp.int32, (1, K), 1) * T
    kTc = jnp.minimum(kT, total_i - 1).astype(jnp.float32)
    ends_f = off_f + pc_f
    te = jnp.zeros((1, K), jnp.int32)
    for e in range(E):
        end_e = lax.slice(ends_f, (0, e), (1, e + 1))
        te = te + (kTc >= end_e).astype(jnp.int32)
    te_ref[...] = te
    ta_ref[...] = (kT < total_i).astype(jnp.int32)


def _mlp_body(te_ref, ta_ref, ps_ref, x_ref, w1_ref, b1_ref, w2_ref, b2_ref,
              o_ref):
    k = pl.program_id(0)

    @pl.when(ta_ref[k] == 1)
    def _():
        x = x_ref[...]
        h = lax.dot_general(
            x, w1_ref[0], (((1,), (1,)), ((), ())),
            preferred_element_type=jnp.float32) + b1_ref[0]
        h = _gelu(h)
        y = lax.dot_general(
            h, w2_ref[0], (((1,), (1,)), ((), ())),
            preferred_element_type=jnp.float32) + b2_ref[0]
        o_ref[...] = ps_ref[:, :1] * y


@functools.lru_cache(maxsize=None)
def _sc_kernels():
    mesh = plsc.VectorSubcoreMesh(core_axis_name="c", subcore_axis_name="s")
    HF = CH // 2
    scratch = [
        pltpu.VMEM((CH,), jnp.int32),
        pltpu.VMEM((CH, D), jnp.float32),
        pltpu.SemaphoreType.DMA,
        pltpu.SemaphoreType.DMA,
        pltpu.SemaphoreType.DMA,
    ]

    @functools.partial(
        pl.kernel, mesh=mesh,
        out_type=[jax.ShapeDtypeStruct((P, D), jnp.float32),
                  jax.ShapeDtypeStruct((P, 128), jnp.float32)],
        scratch_types=[pltpu.VMEM((2, HF), jnp.int32),
                       pltpu.VMEM((CH, D), jnp.float32),
                       pltpu.SemaphoreType.DMA,
                       pltpu.SemaphoreType.DMA,
                       pltpu.SemaphoreType.DMA,
                       pltpu.VMEM((CH, 128), jnp.float32)])
    def dispatch(x_hbm, pw_hbm, dst_hbm, xs_hbm, ps_hbm, idx2, rows_v,
                 sem0, sem1, semp, pv):
        wid = lax.axis_index("s") * NC + lax.axis_index("c")
        base = wid * CH
        pltpu.sync_copy(dst_hbm.at[pl.ds(base, HF)], idx2.at[0])
        pltpu.sync_copy(dst_hbm.at[pl.ds(base + HF, HF)], idx2.at[1])
        ld0 = pltpu.async_copy(
            x_hbm.at[pl.ds(base, HF)], rows_v.at[pl.ds(0, HF)], sem0)
        ld1 = pltpu.async_copy(
            x_hbm.at[pl.ds(base + HF, HF)], rows_v.at[pl.ds(HF, HF)], sem1)
        ldp = pltpu.async_copy(pw_hbm.at[pl.ds(base, CH)], pv, semp)
        ld0.wait()
        st0 = pltpu.async_copy(
            rows_v.at[pl.ds(0, HF)], xs_hbm.at[idx2.at[0]], sem0)
        ld1.wait()
        st1 = pltpu.async_copy(
            rows_v.at[pl.ds(HF, HF)], xs_hbm.at[idx2.at[1]], sem1)
        ldp.wait()
        stp0 = pltpu.async_copy(
            pv.at[pl.ds(0, HF)], ps_hbm.at[idx2.at[0]], semp)
        st0.wait()
        stp1 = pltpu.async_copy(
            pv.at[pl.ds(HF, HF)], ps_hbm.at[idx2.at[1]], semp)
        st1.wait()
        stp0.wait()
        stp1.wait()

    @functools.partial(
        pl.kernel, mesh=mesh,
        out_type=jax.ShapeDtypeStruct((S, D), jnp.float32),
        scratch_types=scratch)
    def combine(ys_hbm, dst_hbm, out_hbm, idx_v, rows_v, sem0, sem1, sem2):
        wid = lax.axis_index("s") * NC + lax.axis_index("c")
        base = wid * CH
        pltpu.sync_copy(dst_hbm.at[pl.ds(base, CH)], idx_v)
        g0 = pltpu.async_copy(
            ys_hbm.at[idx_v.at[pl.ds(0, HF)]], rows_v.at[pl.ds(0, HF)], sem0)
        g1 = pltpu.async_copy(
            ys_hbm.at[idx_v.at[pl.ds(HF, HF)]], rows_v.at[pl.ds(HF, HF)],
            sem1)
        g0.wait()
        s0 = pltpu.async_copy(
            rows_v.at[pl.ds(0, HF)], out_hbm.at[pl.ds(base, HF)], sem0)
        g1.wait()
        s1 = pltpu.async_copy(
            rows_v.at[pl.ds(HF, HF)], out_hbm.at[pl.ds(base + HF, HF)], sem1)
        s0.wait()
        s1.wait()

    return dispatch, combine


def _dispatch_sc(x, pw, dst):
    return _sc_kernels()[0](x, pw, dst)


def _combine_sc(ys, dst):
    return _sc_kernels()[1](ys, dst)


def _router(x, Wr):
    return pl.pallas_call(
        _router_body,
        out_shape=[
            jax.ShapeDtypeStruct((S, 128), jnp.float32),
            jax.ShapeDtypeStruct((S, 1), jnp.int32),
            jax.ShapeDtypeStruct((1, K), jnp.int32),
            jax.ShapeDtypeStruct((1, K), jnp.int32),
        ],
    )(x, Wr)


JH = 4
H2 = H // JH


def _grouped_mlp(te, ta, ps, xs, fc1_w, fc1_b, fc2_w, fc2_b):
    grid_spec = pltpu.PrefetchScalarGridSpec(
        num_scalar_prefetch=2,
        grid=(K,),
        in_specs=[
            pl.BlockSpec((T, 128), lambda k, te, ta: (k, 0)),
            pl.BlockSpec((T, D), lambda k, te, ta: (k, 0)),
            pl.BlockSpec((1, H, D), lambda k, te, ta: (te[k], 0, 0)),
            pl.BlockSpec((1, 1, H), lambda k, te, ta: (te[k], 0, 0)),
            pl.BlockSpec((1, D, H), lambda k, te, ta: (te[k], 0, 0)),
            pl.BlockSpec((1, 1, D), lambda k, te, ta: (te[k], 0, 0)),
        ],
        out_specs=pl.BlockSpec((T, D), lambda k, te, ta: (k, 0)),
    )
    return pl.pallas_call(
        _mlp_body,
        grid_spec=grid_spec,
        out_shape=jax.ShapeDtypeStruct((P, D), jnp.float32),
        compiler_params=pltpu.CompilerParams(
            dimension_semantics=("arbitrary",)),
    )(te, ta, ps, xs, fc1_w, fc1_b.reshape(E, 1, H), fc2_w,
      fc2_b.reshape(E, 1, D))


def kernel(hidden_states, Wr, fc1_w, fc1_b, fc2_w, fc2_b):
    B = hidden_states.shape[0]
    x = hidden_states.reshape(S, D)
    pw, dst2d, te2d, ta2d = _router(x, Wr)
    dst = dst2d.reshape(S)
    xs, ps = _dispatch_sc(x, pw, dst)
    ys = _grouped_mlp(te2d.reshape(K), ta2d.reshape(K), ps, xs,
                      fc1_w, fc1_b, fc2_w, fc2_b)
    y = _combine_sc(ys, dst)
    return y.reshape(B, S, D)


# speedup vs baseline: 1.7743x; 1.0077x over previous
import functools

import jax
import jax.numpy as jnp
from jax import lax
from jax.experimental import pallas as pl
from jax.experimental.pallas import tpu as pltpu
from jax.experimental.pallas import tpu_sc as plsc

D = 768
E = 8
H = 4 * D
S = 2048
T = 256
K = 15
P = K * T
C = 512
NCH = S // C

NC = 2
NS = 16
NW = NC * NS
CH = S // NW


def _gelu(x):
    return x * 0.5 * (1.0 + lax.erf(x * 0.7071067811865476))


def _router_body(x_ref, wr_ref, probs_ref, dst_ref, te_ref, ta_ref, xi_ref):
    logits = lax.dot_general(
        x_ref[...], wr_ref[...], (((1,), (1,)), ((), ())),
        preferred_element_type=jnp.float32)
    m = jnp.max(logits, axis=-1, keepdims=True)
    ex = jnp.exp(logits - m)
    p = ex / jnp.sum(ex, axis=-1, keepdims=True)
    probs_ref[...] = jnp.broadcast_to(jnp.max(p, axis=-1, keepdims=True),
                                      (S, 128))
    eidx = jnp.argmax(p, axis=-1).astype(jnp.int32).reshape(-1, 1)

    lane = lax.broadcasted_iota(jnp.int32, (S, E), 1)
    onehot = (lane == eidx).astype(jnp.float32)

    counts = jnp.sum(onehot, axis=0, keepdims=True)
    pc_i = ((counts.astype(jnp.int32) + T - 1) // T) * T
    pc_f = pc_i.astype(jnp.float32)
    er = lax.broadcasted_iota(jnp.int32, (E, E), 0)
    ec = lax.broadcasted_iota(jnp.int32, (E, E), 1)
    strict8 = (er < ec).astype(jnp.float32)
    off_f = lax.dot_general(pc_f, strict8, (((1,), (0,)), ((), ())),
                            preferred_element_type=jnp.float32)
    total_f = off_f[:, E - 1:E] + pc_f[:, E - 1:E]
    total_i = total_f.astype(jnp.int32)

    rr = lax.broadcasted_iota(jnp.int32, (C, C), 0)
    cc = lax.broadcasted_iota(jnp.int32, (C, C), 1)
    trilC = (cc < rr).astype(jnp.float32)

    carry = jnp.zeros((1, E), jnp.float32)
    dst_parts = []
    for n in range(NCH):
        oh_n = lax.slice(onehot, (n * C, 0), ((n + 1) * C, E))
        within = lax.dot_general(trilC, oh_n, (((1,), (0,)), ((), ())),
                                 preferred_element_type=jnp.float32)
        slot = jnp.sum((within + carry + off_f) * oh_n, axis=-1,
                       keepdims=True)
        dst_parts.append(slot)
        carry = carry + jnp.sum(oh_n, axis=0, keepdims=True)
    dst_ref[...] = jnp.concatenate(dst_parts, axis=0).astype(jnp.int32)

    kT = lax.broadcasted_iota(jnp.int32, (1, K), 1) * T
    kTc = jnp.minimum(kT, total_i - 1).astype(jnp.float32)
    ends_f = off_f + pc_f
    te = jnp.zeros((1, K), jnp.int32)
    for e in range(E):
        end_e = lax.slice(ends_f, (0, e), (1, e + 1))
        te = te + (kTc >= end_e).astype(jnp.int32)
    te_ref[...] = te
    ta_ref[...] = (kT < total_i).astype(jnp.int32)
    k_iota = lax.broadcasted_iota(jnp.int32, (1, K), 1)
    xi_ref[...] = jnp.minimum(k_iota, total_i // T - 1)


def _mlp_body(te_ref, ta_ref, xi_ref, ps_ref, x_ref, w1_ref, b1_ref, w2_ref,
              b2_ref, o_ref):
    k = pl.program_id(0)

    @pl.when(ta_ref[k] == 1)
    def _():
        x = x_ref[...]
        h = lax.dot_general(
            x, w1_ref[0], (((1,), (1,)), ((), ())),
            preferred_element_type=jnp.float32) + b1_ref[0]
        h = _gelu(h)
        y = lax.dot_general(
            h, w2_ref[0], (((1,), (1,)), ((), ())),
            preferred_element_type=jnp.float32) + b2_ref[0]
        o_ref[...] = ps_ref[:, :1] * y


@functools.lru_cache(maxsize=None)
def _sc_kernels():
    mesh = plsc.VectorSubcoreMesh(core_axis_name="c", subcore_axis_name="s")
    HF = CH // 2
    scratch = [
        pltpu.VMEM((CH,), jnp.int32),
        pltpu.VMEM((CH, D), jnp.float32),
        pltpu.SemaphoreType.DMA,
        pltpu.SemaphoreType.DMA,
        pltpu.SemaphoreType.DMA,
    ]

    @functools.partial(
        pl.kernel, mesh=mesh,
        out_type=[jax.ShapeDtypeStruct((P, D), jnp.float32),
                  jax.ShapeDtypeStruct((P, 128), jnp.float32)],
        scratch_types=[pltpu.VMEM((2, HF), jnp.int32),
                       pltpu.VMEM((CH, D), jnp.float32),
                       pltpu.SemaphoreType.DMA,
                       pltpu.SemaphoreType.DMA,
                       pltpu.SemaphoreType.DMA,
                       pltpu.VMEM((CH, 128), jnp.float32)])
    def dispatch(x_hbm, pw_hbm, dst_hbm, xs_hbm, ps_hbm, idx2, rows_v,
                 sem0, sem1, semp, pv):
        wid = lax.axis_index("s") * NC + lax.axis_index("c")
        base = wid * CH
        pltpu.sync_copy(dst_hbm.at[pl.ds(base, HF)], idx2.at[0])
        pltpu.sync_copy(dst_hbm.at[pl.ds(base + HF, HF)], idx2.at[1])
        ld0 = pltpu.async_copy(
            x_hbm.at[pl.ds(base, HF)], rows_v.at[pl.ds(0, HF)], sem0)
        ld1 = pltpu.async_copy(
            x_hbm.at[pl.ds(base + HF, HF)], rows_v.at[pl.ds(HF, HF)], sem1)
        ldp = pltpu.async_copy(pw_hbm.at[pl.ds(base, CH)], pv, semp)
        ld0.wait()
        st0 = pltpu.async_copy(
            rows_v.at[pl.ds(0, HF)], xs_hbm.at[idx2.at[0]], sem0)
        ld1.wait()
        st1 = pltpu.async_copy(
            rows_v.at[pl.ds(HF, HF)], xs_hbm.at[idx2.at[1]], sem1)
        ldp.wait()
        stp0 = pltpu.async_copy(
            pv.at[pl.ds(0, HF)], ps_hbm.at[idx2.at[0]], semp)
        st0.wait()
        stp1 = pltpu.async_copy(
            pv.at[pl.ds(HF, HF)], ps_hbm.at[idx2.at[1]], semp)
        st1.wait()
        stp0.wait()
        stp1.wait()

    @functools.partial(
        pl.kernel, mesh=mesh,
        out_type=jax.ShapeDtypeStruct((S, D), jnp.float32),
        scratch_types=scratch)
    def combine(ys_hbm, dst_hbm, out_hbm, idx_v, rows_v, sem0, sem1, sem2):
        wid = lax.axis_index("s") * NC + lax.axis_index("c")
        base = wid * CH
        pltpu.sync_copy(dst_hbm.at[pl.ds(base, CH)], idx_v)
        g0 = pltpu.async_copy(
            ys_hbm.at[idx_v.at[pl.ds(0, HF)]], rows_v.at[pl.ds(0, HF)], sem0)
        g1 = pltpu.async_copy(
            ys_hbm.at[idx_v.at[pl.ds(HF, HF)]], rows_v.at[pl.ds(HF, HF)],
            sem1)
        g0.wait()
        s0 = pltpu.async_copy(
            rows_v.at[pl.ds(0, HF)], out_hbm.at[pl.ds(base, HF)], sem0)
        g1.wait()
        s1 = pltpu.async_copy(
            rows_v.at[pl.ds(HF, HF)], out_hbm.at[pl.ds(base + HF, HF)], sem1)
        s0.wait()
        s1.wait()

    return dispatch, combine


def _dispatch_sc(x, pw, dst):
    return _sc_kernels()[0](x, pw, dst)


def _combine_sc(ys, dst):
    return _sc_kernels()[1](ys, dst)


def _router(x, Wr):
    return pl.pallas_call(
        _router_body,
        out_shape=[
            jax.ShapeDtypeStruct((S, 128), jnp.float32),
            jax.ShapeDtypeStruct((S, 1), jnp.int32),
            jax.ShapeDtypeStruct((1, K), jnp.int32),
            jax.ShapeDtypeStruct((1, K), jnp.int32),
            jax.ShapeDtypeStruct((1, K), jnp.int32),
        ],
    )(x, Wr)


JH = 4
H2 = H // JH


def _grouped_mlp(te, ta, xi, ps, xs, fc1_w, fc1_b, fc2_w, fc2_b):
    grid_spec = pltpu.PrefetchScalarGridSpec(
        num_scalar_prefetch=3,
        grid=(K,),
        in_specs=[
            pl.BlockSpec((T, 128), lambda k, te, ta, xi: (xi[k], 0)),
            pl.BlockSpec((T, D), lambda k, te, ta, xi: (xi[k], 0)),
            pl.BlockSpec((1, H, D), lambda k, te, ta, xi: (te[k], 0, 0)),
            pl.BlockSpec((1, 1, H), lambda k, te, ta, xi: (te[k], 0, 0)),
            pl.BlockSpec((1, D, H), lambda k, te, ta, xi: (te[k], 0, 0)),
            pl.BlockSpec((1, 1, D), lambda k, te, ta, xi: (te[k], 0, 0)),
        ],
        out_specs=pl.BlockSpec((T, D), lambda k, te, ta, xi: (k, 0)),
    )
    return pl.pallas_call(
        _mlp_body,
        grid_spec=grid_spec,
        out_shape=jax.ShapeDtypeStruct((P, D), jnp.float32),
        compiler_params=pltpu.CompilerParams(
            dimension_semantics=("arbitrary",)),
    )(te, ta, xi, ps, xs, fc1_w, fc1_b.reshape(E, 1, H), fc2_w,
      fc2_b.reshape(E, 1, D))


def kernel(hidden_states, Wr, fc1_w, fc1_b, fc2_w, fc2_b):
    B = hidden_states.shape[0]
    x = hidden_states.reshape(S, D)
    pw, dst2d, te2d, ta2d, xi2d = _router(x, Wr)
    dst = dst2d.reshape(S)
    xs, ps = _dispatch_sc(x, pw, dst)
    ys = _grouped_mlp(te2d.reshape(K), ta2d.reshape(K), xi2d.reshape(K), ps,
                      xs, fc1_w, fc1_b, fc2_w, fc2_b)
    y = _combine_sc(ys, dst)
    return y.reshape(B, S, D)


# speedup vs baseline: 1.7994x; 1.0141x over previous
import functools

import jax
import jax.numpy as jnp
from jax import lax
from jax.experimental import pallas as pl
from jax.experimental.pallas import tpu as pltpu
from jax.experimental.pallas import tpu_sc as plsc

D = 768
E = 8
H = 4 * D
S = 2048
T = 256
K = 15
P = K * T
C = 512
NCH = S // C

NC = 2
NS = 16
NW = NC * NS
CH = S // NW


def _gelu(x):
    return x * 0.5 * (1.0 + lax.erf(x * 0.7071067811865476))


def _router_body(x_ref, wr_ref, probs_ref, dst_ref, te_ref, ta_ref, xi_ref):
    logits = lax.dot_general(
        x_ref[...], wr_ref[...], (((1,), (1,)), ((), ())),
        preferred_element_type=jnp.float32)
    m = jnp.max(logits, axis=-1, keepdims=True)
    ex = jnp.exp(logits - m)
    p = ex / jnp.sum(ex, axis=-1, keepdims=True)
    probs_ref[...] = jnp.broadcast_to(jnp.max(p, axis=-1, keepdims=True),
                                      (S, 128))
    eidx = jnp.argmax(p, axis=-1).astype(jnp.int32).reshape(-1, 1)

    lane = lax.broadcasted_iota(jnp.int32, (S, E), 1)
    onehot = (lane == eidx).astype(jnp.float32)

    counts = jnp.sum(onehot, axis=0, keepdims=True)
    pc_i = ((counts.astype(jnp.int32) + T - 1) // T) * T
    pc_f = pc_i.astype(jnp.float32)
    er = lax.broadcasted_iota(jnp.int32, (E, E), 0)
    ec = lax.broadcasted_iota(jnp.int32, (E, E), 1)
    strict8 = (er < ec).astype(jnp.float32)
    off_f = lax.dot_general(pc_f, strict8, (((1,), (0,)), ((), ())),
                            preferred_element_type=jnp.float32)
    total_f = off_f[:, E - 1:E] + pc_f[:, E - 1:E]
    total_i = total_f.astype(jnp.int32)

    rr = lax.broadcasted_iota(jnp.int32, (C, C), 0)
    cc = lax.broadcasted_iota(jnp.int32, (C, C), 1)
    trilC = (cc < rr).astype(jnp.float32)

    carry = jnp.zeros((1, E), jnp.float32)
    dst_parts = []
    for n in range(NCH):
        oh_n = lax.slice(onehot, (n * C, 0), ((n + 1) * C, E))
        within = lax.dot_general(trilC, oh_n, (((1,), (0,)), ((), ())),
                                 preferred_element_type=jnp.float32)
        slot = jnp.sum((within + carry + off_f) * oh_n, axis=-1,
                       keepdims=True)
        dst_parts.append(slot)
        carry = carry + jnp.sum(oh_n, axis=0, keepdims=True)
    dst_ref[...] = jnp.concatenate(dst_parts, axis=0).astype(jnp.int32)

    kT = lax.broadcasted_iota(jnp.int32, (1, K), 1) * T
    kTc = jnp.minimum(kT, total_i - 1).astype(jnp.float32)
    ends_f = off_f + pc_f
    te = jnp.zeros((1, K), jnp.int32)
    for e in range(E):
        end_e = lax.slice(ends_f, (0, e), (1, e + 1))
        te = te + (kTc >= end_e).astype(jnp.int32)
    te_ref[...] = te
    ta_ref[...] = (kT < total_i).astype(jnp.int32)
    k_iota = lax.broadcasted_iota(jnp.int32, (1, K), 1)
    xi_ref[...] = jnp.minimum(k_iota, total_i // T - 1)


def _mlp_body(te_ref, ta_ref, xi_ref, ps_ref, x_ref, w1_ref, b1_ref, w2_ref,
              b2_ref, o_ref):
    k = pl.program_id(0)

    @pl.when(ta_ref[k] == 1)
    def _():
        x = x_ref[...]
        h = lax.dot_general(
            x, w1_ref[0], (((1,), (1,)), ((), ())),
            preferred_element_type=jnp.float32) + b1_ref[0]
        h = _gelu(h)
        y = lax.dot_general(
            h, w2_ref[0], (((1,), (1,)), ((), ())),
            preferred_element_type=jnp.float32) + b2_ref[0]
        o_ref[...] = ps_ref[:, :1] * y


@functools.lru_cache(maxsize=None)
def _sc_kernels():
    mesh = plsc.VectorSubcoreMesh(core_axis_name="c", subcore_axis_name="s")
    HF = CH // 2
    scratch = [
        pltpu.VMEM((CH,), jnp.int32),
        pltpu.VMEM((CH, D), jnp.float32),
        pltpu.SemaphoreType.DMA,
        pltpu.SemaphoreType.DMA,
        pltpu.SemaphoreType.DMA,
    ]

    @functools.partial(
        pl.kernel, mesh=mesh,
        out_type=[jax.ShapeDtypeStruct((P, D), jnp.float32),
                  jax.ShapeDtypeStruct((P, 128), jnp.float32)],
        scratch_types=[pltpu.VMEM((2, HF), jnp.int32),
                       pltpu.VMEM((CH, D), jnp.float32),
                       pltpu.SemaphoreType.DMA,
                       pltpu.SemaphoreType.DMA,
                       pltpu.SemaphoreType.DMA,
                       pltpu.VMEM((CH, 128), jnp.float32)])
    def dispatch(x_hbm, pw_hbm, dst_hbm, xs_hbm, ps_hbm, idx2, rows_v,
                 sem0, sem1, semp, pv):
        wid = lax.axis_index("s") * NC + lax.axis_index("c")
        base = wid * CH
        pltpu.sync_copy(dst_hbm.at[pl.ds(base, HF)], idx2.at[0])
        pltpu.sync_copy(dst_hbm.at[pl.ds(base + HF, HF)], idx2.at[1])
        ld0 = pltpu.async_copy(
            x_hbm.at[pl.ds(base, HF)], rows_v.at[pl.ds(0, HF)], sem0)
        ld1 = pltpu.async_copy(
            x_hbm.at[pl.ds(base + HF, HF)], rows_v.at[pl.ds(HF, HF)], sem1)
        ldp = pltpu.async_copy(pw_hbm.at[pl.ds(base, CH)], pv, semp)
        ld0.wait()
        st0 = pltpu.async_copy(
            rows_v.at[pl.ds(0, HF)], xs_hbm.at[idx2.at[0]], sem0)
        ld1.wait()
        st1 = pltpu.async_copy(
            rows_v.at[pl.ds(HF, HF)], xs_hbm.at[idx2.at[1]], sem1)
        ldp.wait()
        stp0 = pltpu.async_copy(
            pv.at[pl.ds(0, HF)], ps_hbm.at[idx2.at[0]], semp)
        st0.wait()
        stp1 = pltpu.async_copy(
            pv.at[pl.ds(HF, HF)], ps_hbm.at[idx2.at[1]], semp)
        st1.wait()
        stp0.wait()
        stp1.wait()

    @functools.partial(
        pl.kernel, mesh=mesh,
        out_type=jax.ShapeDtypeStruct((S, D), jnp.float32),
        scratch_types=scratch)
    def combine(ys_hbm, dst_hbm, out_hbm, idx_v, rows_v, sem0, sem1, sem2):
        wid = lax.axis_index("s") * NC + lax.axis_index("c")
        base = wid * CH
        pltpu.sync_copy(dst_hbm.at[pl.ds(base, CH)], idx_v)
        g0 = pltpu.async_copy(
            ys_hbm.at[idx_v.at[pl.ds(0, HF)]], rows_v.at[pl.ds(0, HF)], sem0)
        g1 = pltpu.async_copy(
            ys_hbm.at[idx_v.at[pl.ds(HF, HF)]], rows_v.at[pl.ds(HF, HF)],
            sem1)
        g0.wait()
        s0 = pltpu.async_copy(
            rows_v.at[pl.ds(0, HF)], out_hbm.at[pl.ds(base, HF)], sem0)
        g1.wait()
        s1 = pltpu.async_copy(
            rows_v.at[pl.ds(HF, HF)], out_hbm.at[pl.ds(base + HF, HF)], sem1)
        s0.wait()
        s1.wait()

    return dispatch, combine


def _dispatch_sc(x, pw, dst):
    return _sc_kernels()[0](x, pw, dst)


def _combine_sc(ys, dst):
    return _sc_kernels()[1](ys, dst)


def _router(x, Wr):
    return pl.pallas_call(
        _router_body,
        out_shape=[
            jax.ShapeDtypeStruct((S, 128), jnp.float32),
            jax.ShapeDtypeStruct((S, 1), jnp.int32),
            jax.ShapeDtypeStruct((1, K), jnp.int32),
            jax.ShapeDtypeStruct((1, K), jnp.int32),
            jax.ShapeDtypeStruct((1, K), jnp.int32),
        ],
    )(x, Wr)


JH = 4
H2 = H // JH


def _grouped_mlp(te, ta, xi, ps, xs, fc1_w, fc1_b, fc2_w, fc2_b):
    grid_spec = pltpu.PrefetchScalarGridSpec(
        num_scalar_prefetch=3,
        grid=(K,),
        in_specs=[
            pl.BlockSpec((T, 128), lambda k, te, ta, xi: (xi[k], 0)),
            pl.BlockSpec((T, D), lambda k, te, ta, xi: (xi[k], 0)),
            pl.BlockSpec((1, H, D), lambda k, te, ta, xi: (te[k], 0, 0)),
            pl.BlockSpec((1, 1, H), lambda k, te, ta, xi: (te[k], 0, 0)),
            pl.BlockSpec((1, D, H), lambda k, te, ta, xi: (te[k], 0, 0)),
            pl.BlockSpec((1, 1, D), lambda k, te, ta, xi: (te[k], 0, 0)),
        ],
        out_specs=pl.BlockSpec((T, D), lambda k, te, ta, xi: (xi[k], 0)),
    )
    return pl.pallas_call(
        _mlp_body,
        grid_spec=grid_spec,
        out_shape=jax.ShapeDtypeStruct((P, D), jnp.float32),
        compiler_params=pltpu.CompilerParams(
            dimension_semantics=("arbitrary",)),
    )(te, ta, xi, ps, xs, fc1_w, fc1_b.reshape(E, 1, H), fc2_w,
      fc2_b.reshape(E, 1, D))


def kernel(hidden_states, Wr, fc1_w, fc1_b, fc2_w, fc2_b):
    B = hidden_states.shape[0]
    x = hidden_states.reshape(S, D)
    pw, dst2d, te2d, ta2d, xi2d = _router(x, Wr)
    dst = dst2d.reshape(S)
    xs, ps = _dispatch_sc(x, pw, dst)
    ys = _grouped_mlp(te2d.reshape(K), ta2d.reshape(K), xi2d.reshape(K), ps,
                      xs, fc1_w, fc1_b, fc2_w, fc2_b)
    y = _combine_sc(ys, dst)
    return y.reshape(B, S, D)
